# baseline (speedup 1.0000x reference)
import numpy as np
import jax
import jax.numpy as jnp
from jax import lax
from jax.experimental import pallas as pl
from jax.experimental.pallas import tpu as pltpu
from jax.experimental.pallas import tpu_sc as plsc

_B, _S = 2, 8192
_ROT = 128
_THETA = 1000000.0
_MAXP = 32768
_N = _B * _S

_invf = (1.0 / (_THETA ** (np.arange(0, _ROT, 2, dtype=np.float32) / np.float32(_ROT)))).astype(np.float32)
_tab64 = (np.arange(_MAXP, dtype=np.float32)[:, None] * _invf[None, :]).astype(np.float32)
_TABLE = np.concatenate([_tab64, _tab64], axis=1)

_NC, _NS = 2, 16
_NW = _NC * _NS
_BPW = _N // _NW


def _gather_body(table_hbm, idx_hbm, out_hbm, idx_v, rows_v, sem):
    wid = lax.axis_index("s") * _NC + lax.axis_index("c")
    base = wid * _BPW
    pltpu.sync_copy(idx_hbm.at[pl.ds(base, _BPW)], idx_v)
    pltpu.async_copy(table_hbm.at[idx_v], rows_v, sem).wait()
    pltpu.sync_copy(rows_v, out_hbm.at[pl.ds(base, _BPW)])


def _trig_body(emb_ref, cos_ref, sin_ref):
    e = emb_ref[...]
    cos_ref[...] = jnp.cos(e)
    sin_ref[...] = jnp.sin(e)


def kernel(x, position_ids):
    idx = position_ids.reshape(_N)
    table = jnp.asarray(_TABLE)
    mesh = plsc.VectorSubcoreMesh(core_axis_name="c", subcore_axis_name="s")
    gather = pl.kernel(
        _gather_body,
        mesh=mesh,
        out_type=jax.ShapeDtypeStruct((_N, _ROT), jnp.float32),
        scratch_types=[
            pltpu.VMEM((_BPW,), jnp.int32),
            pltpu.VMEM((_BPW, _ROT), jnp.float32),
            pltpu.SemaphoreType.DMA,
        ],
    )
    emb = gather(table, idx)
    cos, sin = pl.pallas_call(
        _trig_body,
        grid=(8,),
        in_specs=[pl.BlockSpec((_N // 8, _ROT), lambda i: (i, 0))],
        out_specs=[pl.BlockSpec((_N // 8, _ROT), lambda i: (i, 0))] * 2,
        out_shape=[jax.ShapeDtypeStruct((_N, _ROT), jnp.float32)] * 2,
    )(emb)
    dt = x.dtype
    return (cos.reshape(_B, _S, _ROT).astype(dt), sin.reshape(_B, _S, _ROT).astype(dt))


# speedup vs baseline: 2.8377x; 2.8377x over previous
import jax
import jax.numpy as jnp
from jax.experimental import pallas as pl

_B, _S = 2, 8192
_HALF, _ROT = 64, 128
_THETA = 1000000.0
_N = _B * _S
_GRID = 8
_ROWS = 16
_BLK = _ROWS * 128


def _rope_body(pos_ref, cos_ref, sin_ref):
    jcol = jax.lax.broadcasted_iota(jnp.int32, (_HALF, 1), 0).astype(jnp.float32)
    inv_freq_col = 1.0 / (_THETA ** (2.0 * jcol / _ROT))
    for r in range(_ROWS):
        p = pos_ref[0, r, :].astype(jnp.float32)
        pt = jnp.broadcast_to(p.reshape(1, 128), (_HALF, 128))
        ft = pt * inv_freq_col
        ct = jnp.cos(ft).T
        st = jnp.sin(ft).T
        cos_ref[pl.ds(r * 128, 128), 0:_HALF] = ct
        cos_ref[pl.ds(r * 128, 128), _HALF:_ROT] = ct
        sin_ref[pl.ds(r * 128, 128), 0:_HALF] = st
        sin_ref[pl.ds(r * 128, 128), _HALF:_ROT] = st


def kernel(x, position_ids):
    pos = position_ids.reshape(_GRID, _ROWS, 128)
    cos, sin = pl.pallas_call(
        _rope_body,
        grid=(_GRID,),
        in_specs=[pl.BlockSpec((1, _ROWS, 128), lambda i: (i, 0, 0))],
        out_specs=[pl.BlockSpec((_BLK, _ROT), lambda i: (i, 0))] * 2,
        out_shape=[jax.ShapeDtypeStruct((_N, _ROT), jnp.float32)] * 2,
    )(pos)
    dt = x.dtype
    return (cos.reshape(_B, _S, _ROT).astype(dt), sin.reshape(_B, _S, _ROT).astype(dt))


# speedup vs baseline: 4.7476x; 1.6730x over previous
import jax
import jax.numpy as jnp
from jax.experimental import pallas as pl

_B, _S = 2, 8192
_HALF, _ROT = 64, 128
_THETA = 1000000.0
_N = _B * _S
_GRID = 8
_ROWS = 16
_BLK = _ROWS * 128


_TWO_OVER_PI = 0.6366197723675814
_C1 = 1.5703125
_C2 = 4.838267948966e-04
_SIGN = -2147483648


def _sincos(x):
    t = x * _TWO_OVER_PI
    qi = (t + 0.5).astype(jnp.int32)
    qf = qi.astype(jnp.float32)
    r = (x - qf * _C1) - qf * _C2
    r2 = r * r
    ps = 8.3333333e-3 + r2 * (-1.9841270e-4)
    ps = -0.16666667 + r2 * ps
    s = r + (r * r2) * ps
    pc = 4.1666667e-2 + r2 * (-1.3888889e-3)
    pc = -0.5 + r2 * pc
    c = 1.0 + r2 * pc
    swap = (qi & 1) == 1
    sin_pre = jnp.where(swap, c, s)
    cos_pre = jnp.where(swap, s, c)
    sin_sign = (qi << 30) & _SIGN
    cos_sign = ((qi + 1) << 30) & _SIGN
    sin_out = jax.lax.bitcast_convert_type(
        jax.lax.bitcast_convert_type(sin_pre, jnp.int32) ^ sin_sign, jnp.float32)
    cos_out = jax.lax.bitcast_convert_type(
        jax.lax.bitcast_convert_type(cos_pre, jnp.int32) ^ cos_sign, jnp.float32)
    return sin_out, cos_out


def _rope_body(pos_ref, cos_ref, sin_ref):
    jcol = jax.lax.broadcasted_iota(jnp.int32, (_HALF, 1), 0).astype(jnp.float32)
    inv_freq_col = 1.0 / (_THETA ** (2.0 * jcol / _ROT))
    for r in range(_ROWS):
        p = pos_ref[0, r, :].astype(jnp.float32)
        pt = jnp.broadcast_to(p.reshape(1, 128), (_HALF, 128))
        ft = pt * inv_freq_col
        sft, cft = _sincos(ft)
        ct = cft.T
        st = sft.T
        cos_ref[pl.ds(r * 128, 128), 0:_HALF] = ct
        cos_ref[pl.ds(r * 128, 128), _HALF:_ROT] = ct
        sin_ref[pl.ds(r * 128, 128), 0:_HALF] = st
        sin_ref[pl.ds(r * 128, 128), _HALF:_ROT] = st


def kernel(x, position_ids):
    pos = position_ids.reshape(_GRID, _ROWS, 128)
    cos, sin = pl.pallas_call(
        _rope_body,
        grid=(_GRID,),
        in_specs=[pl.BlockSpec((1, _ROWS, 128), lambda i: (i, 0, 0))],
        out_specs=[pl.BlockSpec((_BLK, _ROT), lambda i: (i, 0))] * 2,
        out_shape=[jax.ShapeDtypeStruct((_N, _ROT), jnp.float32)] * 2,
    )(pos)
    dt = x.dtype
    return (cos.reshape(_B, _S, _ROT).astype(dt), sin.reshape(_B, _S, _ROT).astype(dt))


# speedup vs baseline: 4.7577x; 1.0021x over previous
import jax
import jax.numpy as jnp
from jax.experimental import pallas as pl

_B, _S = 2, 8192
_HALF, _ROT = 64, 128
_THETA = 1000000.0
_N = _B * _S
_GRID = 8
_ROWS = 16
_BLK = _ROWS * 128


_TWO_OVER_PI = 0.6366197723675814
_C1 = 1.5703125
_C2 = 4.838267948966e-04
_SIGN = -2147483648


def _sincos(x):
    t = x * _TWO_OVER_PI
    qi = (t + 0.5).astype(jnp.int32)
    qf = qi.astype(jnp.float32)
    r = (x - qf * _C1) - qf * _C2
    r2 = r * r
    ps = 8.3333333e-3 + r2 * (-1.9841270e-4)
    ps = -0.16666667 + r2 * ps
    s = r + (r * r2) * ps
    pc = 4.1666667e-2 + r2 * (-1.3888889e-3)
    pc = -0.5 + r2 * pc
    c = 1.0 + r2 * pc
    swap = (qi & 1) == 1
    sin_pre = jnp.where(swap, c, s)
    cos_pre = jnp.where(swap, s, c)
    sin_sign = (qi << 30) & _SIGN
    cos_sign = ((qi + 1) << 30) & _SIGN
    sin_out = jax.lax.bitcast_convert_type(
        jax.lax.bitcast_convert_type(sin_pre, jnp.int32) ^ sin_sign, jnp.float32)
    cos_out = jax.lax.bitcast_convert_type(
        jax.lax.bitcast_convert_type(cos_pre, jnp.int32) ^ cos_sign, jnp.float32)
    return sin_out, cos_out


def _rope_body(pos_ref, cos_ref, sin_ref):
    jcol = jax.lax.broadcasted_iota(jnp.int32, (_HALF, 1), 0).astype(jnp.float32)
    inv_freq_col = 1.0 / (_THETA ** (2.0 * jcol / _ROT))
    for r in range(_ROWS):
        p = pos_ref[0, r, :].astype(jnp.float32)
        pt = jnp.broadcast_to(p.reshape(1, 128), (_HALF, 128))
        ft = pt * inv_freq_col
        sft, cft = _sincos(ft)
        ct = cft.T
        st = sft.T
        cos_ref[pl.ds(r * 128, 128), :] = jnp.concatenate([ct, ct], axis=-1)
        sin_ref[pl.ds(r * 128, 128), :] = jnp.concatenate([st, st], axis=-1)


def kernel(x, position_ids):
    pos = position_ids.reshape(_GRID, _ROWS, 128)
    cos, sin = pl.pallas_call(
        _rope_body,
        grid=(_GRID,),
        in_specs=[pl.BlockSpec((1, _ROWS, 128), lambda i: (i, 0, 0))],
        out_specs=[pl.BlockSpec((_BLK, _ROT), lambda i: (i, 0))] * 2,
        out_shape=[jax.ShapeDtypeStruct((_N, _ROT), jnp.float32)] * 2,
    )(pos)
    dt = x.dtype
    return (cos.reshape(_B, _S, _ROT).astype(dt), sin.reshape(_B, _S, _ROT).astype(dt))
